# baseline (speedup 1.0000x reference)
import functools
import math

import jax
import jax.numpy as jnp
from jax import lax
from jax.experimental import pallas as pl
from jax.experimental.pallas import tpu as pltpu
from jax.experimental.pallas import tpu_sc as plsc

_BINS = 64
_MIN = -4.0
_MAX = 4.0
_SIGMA = 100.0
_DELTA = (_MAX - _MIN) / _BINS
_SD = _SIGMA * _DELTA
_INV_DELTA = 1.0 / _DELTA

_ROWS = 8
_COLS = 131072
_SEG = (_ROWS * _COLS) // 16
_NV = _SEG // 16

_CHUNK = 16384
_NCH = _SEG // _CHUNK

_R = 1
_PAD = _R
_STRIDE = 72
_HSIZE = 16 * _STRIDE

_EDGE_SCALE = [math.exp(_SD * d) for d in range(-_R, _R + 2)]

_mesh = plsc.VectorSubcoreMesh(core_axis_name="c", subcore_axis_name="s")


@functools.partial(
    pl.kernel,
    mesh=_mesh,
    out_type=jax.ShapeDtypeStruct((32 * _BINS,), jnp.float32),
    scratch_types=[
        pltpu.VMEM((2 * _CHUNK,), jnp.float32),
        pltpu.VMEM((_HSIZE,), jnp.float32),
        pltpu.VMEM((_BINS,), jnp.float32),
        pltpu.SemaphoreType.DMA,
        pltpu.SemaphoreType.DMA,
    ],
    compiler_params=pltpu.CompilerParams(needs_layout_passes=False),
)
def _sc_hists(out_hbm, tgt_hbm, hists_hbm, xbuf, h2d, hrow, sem0, sem1):
    cid = lax.axis_index("c")
    sid = lax.axis_index("s")
    off = sid * _SEG
    sems = (sem0, sem1)

    def start_chunk(g):
        dst = xbuf.at[pl.ds((g % 2) * _CHUNK, _CHUNK)]
        src = pl.ds(off + g * _CHUNK, _CHUNK)

        @pl.when(cid == 0)
        def _():
            pltpu.async_copy(out_hbm.at[src], dst, sems[g % 2])

        @pl.when(cid == 1)
        def _():
            pltpu.async_copy(tgt_hbm.at[src], dst, sems[g % 2])

    def wait_chunk(g):
        pltpu.make_async_copy(
            out_hbm.at[pl.ds(0, _CHUNK)],
            xbuf.at[pl.ds((g % 2) * _CHUNK, _CHUNK)],
            sems[g % 2],
        ).wait()

    start_chunk(0)

    zero = jnp.zeros((16,), jnp.float32)

    def zbody(i, carry):
        h2d[pl.ds(i * 16, 16)] = zero
        return carry

    lax.fori_loop(0, _HSIZE // 16, zbody, 0)

    lane = lax.iota(jnp.int32, 16)
    bases = [lane * _STRIDE + (o + _PAD) for o in range(-_R, _R + 1)]

    exp_sd = math.exp(_SD)

    for g in range(_NCH):
        if g + 1 < _NCH:
            start_chunk(g + 1)
        wait_chunk(g)
        buf_base = (g % 2) * _CHUNK

        @plsc.parallel_loop(buf_base, buf_base + _CHUNK, step=16, unroll=5)
        def _loop(i):
            x = xbuf[pl.ds(i, 16)]
            u = x * _INV_DELTA + (-_MIN * _INV_DELTA)
            uc = jnp.minimum(jnp.maximum(u, 0.0), float(_BINS - 1))
            c = uc.astype(jnp.int32)
            e = jnp.exp((c.astype(jnp.float32) - u) * _SD)
            a = 1.0 / (1.0 + e)
            b = 1.0 / (1.0 + e * exp_sd)
            for k, w in enumerate((1.0 - a, a - b, b)):
                plsc.addupdate_scatter(h2d, [bases[k] + c], w)

    for q in range(_BINS // 16):
        acc = h2d[pl.ds(_PAD + q * 16, 16)]
        for l in range(1, 16):
            acc = acc + h2d[pl.ds(l * _STRIDE + _PAD + q * 16, 16)]
        hrow[pl.ds(q * 16, 16)] = acc

    slot = cid * 16 + (sid % 2) * 8 + sid // 2
    pltpu.sync_copy(hrow, hists_hbm.at[pl.ds(slot * _BINS, _BINS)])


def _tc_loss_body(h_ref, o_ref):
    x = h_ref[...]
    oh = x[0:8] + x[8:16]
    th = x[16:24] + x[24:32]
    n = 1e-07 + jnp.sqrt(jnp.sum(oh * oh, axis=1, keepdims=True))
    loss = jnp.sum(jnp.abs(oh - th) / n) / float(_ROWS * _BINS)
    o_ref[...] = jnp.reshape(loss, (1, 1))


def kernel(output, target):
    hists = _sc_hists(output.reshape(-1), target.reshape(-1))
    loss = pl.pallas_call(
        _tc_loss_body,
        out_shape=jax.ShapeDtypeStruct((1, 1), jnp.float32),
    )(hists.reshape(32, _BINS))
    return loss[0, 0]


# speedup vs baseline: 1.0021x; 1.0021x over previous
import functools
import math

import jax
import jax.numpy as jnp
from jax import lax
from jax.experimental import pallas as pl
from jax.experimental.pallas import tpu as pltpu
from jax.experimental.pallas import tpu_sc as plsc

_BINS = 64
_MIN = -4.0
_MAX = 4.0
_SIGMA = 100.0
_DELTA = (_MAX - _MIN) / _BINS
_SD = _SIGMA * _DELTA
_INV_DELTA = 1.0 / _DELTA

_ROWS = 8
_COLS = 131072
_SEG = (_ROWS * _COLS) // 16
_NV = _SEG // 16

_CHUNK = 8192
_NCH = _SEG // _CHUNK

_R = 1
_PAD = _R
_STRIDE = 72
_HSIZE = 16 * _STRIDE

_EDGE_SCALE = [math.exp(_SD * d) for d in range(-_R, _R + 2)]

_mesh = plsc.VectorSubcoreMesh(core_axis_name="c", subcore_axis_name="s")


@functools.partial(
    pl.kernel,
    mesh=_mesh,
    out_type=jax.ShapeDtypeStruct((32 * _BINS,), jnp.float32),
    scratch_types=[
        pltpu.VMEM((2 * _CHUNK,), jnp.float32),
        pltpu.VMEM((_HSIZE,), jnp.float32),
        pltpu.VMEM((_BINS,), jnp.float32),
        pltpu.SemaphoreType.DMA,
        pltpu.SemaphoreType.DMA,
    ],
    compiler_params=pltpu.CompilerParams(needs_layout_passes=False),
)
def _sc_hists(out_hbm, tgt_hbm, hists_hbm, xbuf, h2d, hrow, sem0, sem1):
    cid = lax.axis_index("c")
    sid = lax.axis_index("s")
    off = sid * _SEG
    sems = (sem0, sem1)

    def start_chunk(g):
        dst = xbuf.at[pl.ds((g % 2) * _CHUNK, _CHUNK)]
        src = pl.ds(off + g * _CHUNK, _CHUNK)

        @pl.when(cid == 0)
        def _():
            pltpu.async_copy(out_hbm.at[src], dst, sems[g % 2])

        @pl.when(cid == 1)
        def _():
            pltpu.async_copy(tgt_hbm.at[src], dst, sems[g % 2])

    def wait_chunk(g):
        pltpu.make_async_copy(
            out_hbm.at[pl.ds(0, _CHUNK)],
            xbuf.at[pl.ds((g % 2) * _CHUNK, _CHUNK)],
            sems[g % 2],
        ).wait()

    start_chunk(0)

    zero = jnp.zeros((16,), jnp.float32)

    def zbody(i, carry):
        h2d[pl.ds(i * 16, 16)] = zero
        return carry

    lax.fori_loop(0, _HSIZE // 16, zbody, 0)

    lane = lax.iota(jnp.int32, 16)
    bases = [lane * _STRIDE + (o + _PAD) for o in range(-_R, _R + 1)]

    exp_sd = math.exp(_SD)

    for g in range(_NCH):
        if g + 1 < _NCH:
            start_chunk(g + 1)
        wait_chunk(g)
        buf_base = (g % 2) * _CHUNK

        @plsc.parallel_loop(buf_base, buf_base + _CHUNK, step=16, unroll=4)
        def _loop(i):
            x = xbuf[pl.ds(i, 16)]
            u = x * _INV_DELTA + (-_MIN * _INV_DELTA)
            uc = jnp.minimum(jnp.maximum(u, 0.0), float(_BINS - 1))
            c = uc.astype(jnp.int32)
            e = jnp.exp((c.astype(jnp.float32) - u) * _SD)
            a = 1.0 / (1.0 + e)
            b = 1.0 / (1.0 + e * exp_sd)
            for k, w in enumerate((1.0 - a, a - b, b)):
                plsc.addupdate_scatter(h2d, [bases[k] + c], w)

    for q in range(_BINS // 16):
        acc = h2d[pl.ds(_PAD + q * 16, 16)]
        for l in range(1, 16):
            acc = acc + h2d[pl.ds(l * _STRIDE + _PAD + q * 16, 16)]
        hrow[pl.ds(q * 16, 16)] = acc

    slot = cid * 16 + (sid % 2) * 8 + sid // 2
    pltpu.sync_copy(hrow, hists_hbm.at[pl.ds(slot * _BINS, _BINS)])


def _tc_loss_body(h_ref, o_ref):
    x = h_ref[...]
    oh = x[0:8] + x[8:16]
    th = x[16:24] + x[24:32]
    n = 1e-07 + jnp.sqrt(jnp.sum(oh * oh, axis=1, keepdims=True))
    loss = jnp.sum(jnp.abs(oh - th) / n) / float(_ROWS * _BINS)
    o_ref[...] = jnp.reshape(loss, (1, 1))


def _tc_dummy_body(x_ref, o_ref):
    x = x_ref[...]
    acc = x
    for _ in range(6):
        acc = jnp.tanh(acc * 1.0001 + x)
    o_ref[...] = acc


def kernel(output, target):
    dummy = pl.pallas_call(
        _tc_dummy_body,
        out_shape=jax.ShapeDtypeStruct((8, 16384), jnp.float32),
    )(output[:, :16384])
    hists = _sc_hists(output.reshape(-1), target.reshape(-1))
    loss = pl.pallas_call(
        _tc_loss_body,
        out_shape=jax.ShapeDtypeStruct((1, 1), jnp.float32),
    )(hists.reshape(32, _BINS))
    return loss[0, 0] + 0.0 * dummy[0, 0]


# speedup vs baseline: 1.0892x; 1.0869x over previous
import functools
import math

import jax
import jax.numpy as jnp
from jax import lax
from jax.experimental import pallas as pl
from jax.experimental.pallas import tpu as pltpu
from jax.experimental.pallas import tpu_sc as plsc

_BINS = 64
_MIN = -4.0
_MAX = 4.0
_SIGMA = 100.0
_DELTA = (_MAX - _MIN) / _BINS
_SD = _SIGMA * _DELTA
_INV_DELTA = 1.0 / _DELTA

_ROWS = 8
_COLS = 131072
_TCOLS = 16384
_SCCOLS = _COLS - _TCOLS
_SEG = _SCCOLS // 2
_NV = _SEG // 16

_CHUNK = 8192
_NCH = _SEG // _CHUNK

_R = 1
_PAD = _R
_STRIDE = 72
_HSIZE = 16 * _STRIDE

_EDGE_SCALE = [math.exp(_SD * d) for d in range(-_R, _R + 2)]

_mesh = plsc.VectorSubcoreMesh(core_axis_name="c", subcore_axis_name="s")


@functools.partial(
    pl.kernel,
    mesh=_mesh,
    out_type=jax.ShapeDtypeStruct((32 * _BINS,), jnp.float32),
    scratch_types=[
        pltpu.VMEM((2 * _CHUNK,), jnp.float32),
        pltpu.VMEM((_HSIZE,), jnp.float32),
        pltpu.VMEM((_BINS,), jnp.float32),
        pltpu.SemaphoreType.DMA,
        pltpu.SemaphoreType.DMA,
    ],
    compiler_params=pltpu.CompilerParams(needs_layout_passes=False),
)
def _sc_hists(out_hbm, tgt_hbm, hists_hbm, xbuf, h2d, hrow, sem0, sem1):
    cid = lax.axis_index("c")
    sid = lax.axis_index("s")
    off = (sid // 2) * _COLS + (sid % 2) * _SEG
    sems = (sem0, sem1)

    def start_chunk(g):
        dst = xbuf.at[pl.ds((g % 2) * _CHUNK, _CHUNK)]
        src = pl.ds(off + g * _CHUNK, _CHUNK)

        @pl.when(cid == 0)
        def _():
            pltpu.async_copy(out_hbm.at[src], dst, sems[g % 2])

        @pl.when(cid == 1)
        def _():
            pltpu.async_copy(tgt_hbm.at[src], dst, sems[g % 2])

    def wait_chunk(g):
        pltpu.make_async_copy(
            out_hbm.at[pl.ds(0, _CHUNK)],
            xbuf.at[pl.ds((g % 2) * _CHUNK, _CHUNK)],
            sems[g % 2],
        ).wait()

    start_chunk(0)

    zero = jnp.zeros((16,), jnp.float32)

    def zbody(i, carry):
        h2d[pl.ds(i * 16, 16)] = zero
        return carry

    lax.fori_loop(0, _HSIZE // 16, zbody, 0)

    lane = lax.iota(jnp.int32, 16)
    bases = [lane * _STRIDE + (o + _PAD) for o in range(-_R, _R + 1)]

    exp_sd = math.exp(_SD)

    for g in range(_NCH):
        if g + 1 < _NCH:
            start_chunk(g + 1)
        wait_chunk(g)
        buf_base = (g % 2) * _CHUNK

        @plsc.parallel_loop(buf_base, buf_base + _CHUNK, step=16, unroll=4)
        def _loop(i):
            x = xbuf[pl.ds(i, 16)]
            u = x * _INV_DELTA + (-_MIN * _INV_DELTA)
            uc = jnp.minimum(jnp.maximum(u, 0.0), float(_BINS - 1))
            c = uc.astype(jnp.int32)
            e = jnp.exp((c.astype(jnp.float32) - u) * _SD)
            a = 1.0 / (1.0 + e)
            b = 1.0 / (1.0 + e * exp_sd)
            for k, w in enumerate((1.0 - a, a - b, b)):
                plsc.addupdate_scatter(h2d, [bases[k] + c], w)

    for q in range(_BINS // 16):
        acc = h2d[pl.ds(_PAD + q * 16, 16)]
        for l in range(1, 16):
            acc = acc + h2d[pl.ds(l * _STRIDE + _PAD + q * 16, 16)]
        hrow[pl.ds(q * 16, 16)] = acc

    slot = cid * 16 + (sid % 2) * 8 + sid // 2
    pltpu.sync_copy(hrow, hists_hbm.at[pl.ds(slot * _BINS, _BINS)])


def _tc_tail_body(o_ref, t_ref, h_ref):
    for a, ref in enumerate((o_ref, t_ref)):
        x = ref[...]
        u125 = x * (_INV_DELTA * _SD) + (-_MIN * _INV_DELTA * _SD)
        g = [jnp.sum(jax.nn.sigmoid(u125 - _SD * e), axis=1, keepdims=True)
             for e in range(_BINS + 1)]
        gm = jnp.concatenate(g, axis=1)
        h_ref[a * 8:(a + 1) * 8, :] = gm[:, :-1] - gm[:, 1:]


def _tc_loss_body(h_ref, tail_ref, o_ref):
    x = h_ref[...]
    tl = tail_ref[...]
    oh = x[0:8] + x[8:16] + tl[0:8]
    th = x[16:24] + x[24:32] + tl[8:16]
    n = 1e-07 + jnp.sqrt(jnp.sum(oh * oh, axis=1, keepdims=True))
    loss = jnp.sum(jnp.abs(oh - th) / n) / float(_ROWS * _BINS)
    o_ref[...] = jnp.reshape(loss, (1, 1))


def kernel(output, target):
    hists = _sc_hists(output.reshape(-1), target.reshape(-1))
    tail = pl.pallas_call(
        _tc_tail_body,
        out_shape=jax.ShapeDtypeStruct((16, _BINS), jnp.float32),
        grid=(1,),
        in_specs=[
            pl.BlockSpec((_ROWS, _TCOLS), lambda i: (0, _COLS // _TCOLS - 1)),
            pl.BlockSpec((_ROWS, _TCOLS), lambda i: (0, _COLS // _TCOLS - 1)),
        ],
        out_specs=pl.BlockSpec((16, _BINS), lambda i: (0, 0)),
    )(output, target)
    loss = pl.pallas_call(
        _tc_loss_body,
        out_shape=jax.ShapeDtypeStruct((1, 1), jnp.float32),
    )(hists.reshape(32, _BINS), tail)
    return loss[0, 0]


# speedup vs baseline: 1.1568x; 1.0620x over previous
import functools
import math

import jax
import jax.numpy as jnp
from jax import lax
from jax.experimental import pallas as pl
from jax.experimental.pallas import tpu as pltpu
from jax.experimental.pallas import tpu_sc as plsc

_BINS = 64
_MIN = -4.0
_MAX = 4.0
_SIGMA = 100.0
_DELTA = (_MAX - _MIN) / _BINS
_SD = _SIGMA * _DELTA
_INV_DELTA = 1.0 / _DELTA

_ROWS = 8
_COLS = 131072
_TCOLS = 32768
_SCCOLS = _COLS - _TCOLS
_SEG = _SCCOLS // 2
_NV = _SEG // 16

_CHUNK = 8192
_NCH = _SEG // _CHUNK

_R = 1
_PAD = _R
_STRIDE = 72
_HSIZE = 16 * _STRIDE

_EDGE_SCALE = [math.exp(_SD * d) for d in range(-_R, _R + 2)]

_mesh = plsc.VectorSubcoreMesh(core_axis_name="c", subcore_axis_name="s")


@functools.partial(
    pl.kernel,
    mesh=_mesh,
    out_type=jax.ShapeDtypeStruct((32 * _BINS,), jnp.float32),
    scratch_types=[
        pltpu.VMEM((2 * _CHUNK,), jnp.float32),
        pltpu.VMEM((_HSIZE,), jnp.float32),
        pltpu.VMEM((_BINS,), jnp.float32),
        pltpu.SemaphoreType.DMA,
        pltpu.SemaphoreType.DMA,
    ],
    compiler_params=pltpu.CompilerParams(needs_layout_passes=False),
)
def _sc_hists(out_hbm, tgt_hbm, hists_hbm, xbuf, h2d, hrow, sem0, sem1):
    cid = lax.axis_index("c")
    sid = lax.axis_index("s")
    off = (sid // 2) * _COLS + (sid % 2) * _SEG
    sems = (sem0, sem1)

    def start_chunk(g):
        dst = xbuf.at[pl.ds((g % 2) * _CHUNK, _CHUNK)]
        src = pl.ds(off + g * _CHUNK, _CHUNK)

        @pl.when(cid == 0)
        def _():
            pltpu.async_copy(out_hbm.at[src], dst, sems[g % 2])

        @pl.when(cid == 1)
        def _():
            pltpu.async_copy(tgt_hbm.at[src], dst, sems[g % 2])

    def wait_chunk(g):
        pltpu.make_async_copy(
            out_hbm.at[pl.ds(0, _CHUNK)],
            xbuf.at[pl.ds((g % 2) * _CHUNK, _CHUNK)],
            sems[g % 2],
        ).wait()

    start_chunk(0)

    zero = jnp.zeros((16,), jnp.float32)

    def zbody(i, carry):
        h2d[pl.ds(i * 16, 16)] = zero
        return carry

    lax.fori_loop(0, _HSIZE // 16, zbody, 0)

    lane = lax.iota(jnp.int32, 16)
    bases = [lane * _STRIDE + (o + _PAD) for o in range(-_R, _R + 1)]

    exp_sd = math.exp(_SD)

    for g in range(_NCH):
        if g + 1 < _NCH:
            start_chunk(g + 1)
        wait_chunk(g)
        buf_base = (g % 2) * _CHUNK

        @plsc.parallel_loop(buf_base, buf_base + _CHUNK, step=16, unroll=4)
        def _loop(i):
            x = xbuf[pl.ds(i, 16)]
            u = x * _INV_DELTA + (-_MIN * _INV_DELTA)
            uc = jnp.minimum(jnp.maximum(u, 0.0), float(_BINS - 1))
            c = uc.astype(jnp.int32)
            e = jnp.exp((c.astype(jnp.float32) - u) * _SD)
            a = 1.0 / (1.0 + e)
            b = 1.0 / (1.0 + e * exp_sd)
            for k, w in enumerate((1.0 - a, a - b, b)):
                plsc.addupdate_scatter(h2d, [bases[k] + c], w)

    for q in range(_BINS // 16):
        acc = h2d[pl.ds(_PAD + q * 16, 16)]
        for l in range(1, 16):
            acc = acc + h2d[pl.ds(l * _STRIDE + _PAD + q * 16, 16)]
        hrow[pl.ds(q * 16, 16)] = acc

    slot = cid * 16 + (sid % 2) * 8 + sid // 2
    pltpu.sync_copy(hrow, hists_hbm.at[pl.ds(slot * _BINS, _BINS)])


def _tc_tail_body(o_ref, t_ref, h_ref):
    for a, ref in enumerate((o_ref, t_ref)):
        x = ref[...]
        uh = x * (_INV_DELTA * _SD * 0.5) + (-_MIN * _INV_DELTA * _SD * 0.5)
        g = [jnp.sum(jnp.tanh(uh - (_SD * 0.5) * e), axis=1, keepdims=True)
             for e in range(_BINS + 1)]
        gm = jnp.concatenate(g, axis=1)
        h_ref[a * 8:(a + 1) * 8, :] = 0.5 * (gm[:, :-1] - gm[:, 1:])


def _tc_loss_body(h_ref, tail_ref, o_ref):
    x = h_ref[...]
    tl = tail_ref[...]
    oh = x[0:8] + x[8:16] + tl[0:8]
    th = x[16:24] + x[24:32] + tl[8:16]
    n = 1e-07 + jnp.sqrt(jnp.sum(oh * oh, axis=1, keepdims=True))
    loss = jnp.sum(jnp.abs(oh - th) / n) / float(_ROWS * _BINS)
    o_ref[...] = jnp.reshape(loss, (1, 1))


def kernel(output, target):
    hists = _sc_hists(output.reshape(-1), target.reshape(-1))
    tail = pl.pallas_call(
        _tc_tail_body,
        out_shape=jax.ShapeDtypeStruct((16, _BINS), jnp.float32),
        grid=(1,),
        in_specs=[
            pl.BlockSpec((_ROWS, _TCOLS), lambda i: (0, _COLS // _TCOLS - 1)),
            pl.BlockSpec((_ROWS, _TCOLS), lambda i: (0, _COLS // _TCOLS - 1)),
        ],
        out_specs=pl.BlockSpec((16, _BINS), lambda i: (0, 0)),
    )(output, target)
    loss = pl.pallas_call(
        _tc_loss_body,
        out_shape=jax.ShapeDtypeStruct((1, 1), jnp.float32),
    )(hists.reshape(32, _BINS), tail)
    return loss[0, 0]
